# baseline (speedup 1.0000x reference)
import functools
import math

import jax
import jax.numpy as jnp
import numpy as np
from jax import lax
from jax.experimental import pallas as pl
from jax.experimental.pallas import tpu as pltpu
from jax.experimental.pallas import tpu_sc as plsc


def _pe_table(max_len, d):
    position = np.arange(max_len, dtype=np.float32)[:, None]
    div_term = np.exp(
        np.arange(0, d, 2, dtype=np.float32) * (-math.log(10000.0) / d)
    )
    pe = np.zeros((max_len, d), dtype=np.float32)
    pe[:, 0::2] = np.sin(position * div_term)
    pe[:, 1::2] = np.cos(position * div_term)
    return pe


def _sc_gather(table, ids_flat, mask_flat_i32):
    N = ids_flat.shape[0]
    V, D = table.shape
    info = plsc.get_sparse_core_info()
    NC, NS, L = info.num_cores, info.num_subcores, info.num_lanes
    NW = NC * NS
    assert N % NW == 0
    b_per_w = N // NW
    K = 32
    NBUF = 4
    assert b_per_w % K == 0
    n_chunks = b_per_w // K
    assert n_chunks % NBUF == 0
    mesh = plsc.VectorSubcoreMesh(core_axis_name="c", subcore_axis_name="s")

    @functools.partial(
        pl.kernel,
        mesh=mesh,
        out_type=jax.ShapeDtypeStruct((N, D), jnp.float32),
        scratch_types=[
            pltpu.VMEM((b_per_w,), jnp.int32),
            pltpu.VMEM((b_per_w,), jnp.int32),
            [pltpu.VMEM((K, D), jnp.float32) for _ in range(NBUF)],
            [pltpu.SemaphoreType.DMA for _ in range(NBUF)],
            [pltpu.SemaphoreType.DMA for _ in range(NBUF)],
        ],
    )
    def gather_kernel(table_hbm, ids_hbm, mask_hbm, out_hbm, idx_v, msk_v,
                      rows, gsems, wsems):
        wid = lax.axis_index("s") * NC + lax.axis_index("c")
        base = wid * b_per_w
        pltpu.sync_copy(ids_hbm.at[pl.ds(base, b_per_w)], idx_v)
        pltpu.sync_copy(mask_hbm.at[pl.ds(base, b_per_w)], msk_v)

        def mask_body(i, _):
            iv = idx_v[pl.ds(i * L, L)]
            mv = msk_v[pl.ds(i * L, L)]
            idx_v[pl.ds(i * L, L)] = iv * (1 - mv)
            return 0

        lax.fori_loop(0, b_per_w // L, mask_body, 0)

        def g_copy(c, b):
            return pltpu.make_async_copy(
                table_hbm.at[idx_v.at[pl.ds(c * K, K)]], rows[b], gsems[b]
            )

        def w_copy(c, b):
            return pltpu.make_async_copy(
                rows[b], out_hbm.at[pl.ds(base + c * K, K)], wsems[b]
            )

        for b in range(NBUF - 1):
            g_copy(b, b).start()

        def group(g, _):
            for b in range(NBUF):
                c = g * NBUF + b
                g_copy(c, b).wait()
                w_copy(c, b).start()
                f = c + NBUF - 1
                fb = (b + NBUF - 1) % NBUF

                @pl.when(f < n_chunks)
                def _():
                    @pl.when(f >= NBUF)
                    def _():
                        w_copy(f - NBUF, fb).wait()

                    g_copy(f, fb).start()
            return 0

        lax.fori_loop(0, n_chunks // NBUF, group, 0)
        for b in range(NBUF):
            c = n_chunks - NBUF + b
            w_copy(c, c % NBUF).wait()

    return gather_kernel(table, ids_flat, mask_flat_i32)


def _tc_fuse(gathered, ids_col, types_col, mask_col, w1, b1, w2, b2, type_emb,
             pe_tiled, gamma, beta):
    N, D = gathered.shape
    H = w1.shape[1]
    T = type_emb.shape[0]
    BLK = pe_tiled.shape[0]
    assert N % BLK == 0

    def body(g_ref, ids_ref, ty_ref, mk_ref, w1_ref, b1_ref, w2_ref, b2_ref,
             te_ref, pe_ref, ga_ref, be_ref, out_ref):
        ids = ids_ref[...]
        h = jnp.maximum(ids * w1_ref[...] + b1_ref[...], 0.0)
        cont = jnp.dot(h, w2_ref[...], preferred_element_type=jnp.float32) + b2_ref[...]
        ty = ty_ref[...]
        onehot = (ty == lax.broadcasted_iota(jnp.int32, (BLK, T), 1)).astype(jnp.float32)
        tvec = jnp.dot(onehot, te_ref[...], preferred_element_type=jnp.float32)
        mk = mk_ref[...]
        emb = jnp.where(mk != 0, cont, g_ref[...]) + tvec + pe_ref[...]
        mean = jnp.mean(emb, axis=-1, keepdims=True)
        var = jnp.mean(jnp.square(emb - mean), axis=-1, keepdims=True)
        out_ref[...] = (emb - mean) * lax.rsqrt(var + 1e-5) * ga_ref[...] + be_ref[...]

    grid = (N // BLK,)
    full = lambda shape: pl.BlockSpec(shape, lambda i: (0,) * len(shape))
    return pl.pallas_call(
        body,
        grid=grid,
        in_specs=[
            pl.BlockSpec((BLK, D), lambda i: (i, 0)),
            pl.BlockSpec((BLK, 1), lambda i: (i, 0)),
            pl.BlockSpec((BLK, 1), lambda i: (i, 0)),
            pl.BlockSpec((BLK, 1), lambda i: (i, 0)),
            full((1, H)),
            full((1, H)),
            full((H, D)),
            full((1, D)),
            full((T, D)),
            full((BLK, D)),
            full((1, D)),
            full((1, D)),
        ],
        out_specs=pl.BlockSpec((BLK, D), lambda i: (i, 0)),
        out_shape=jax.ShapeDtypeStruct((N, D), jnp.float32),
    )(gathered, ids_col, types_col, mask_col, w1, b1, w2, b2, type_emb,
      pe_tiled, gamma, beta)


def kernel(token_ids, token_types, continuous_mask, token_emb, w1, b1, w2, b2,
           type_emb, gamma, beta):
    B, S = token_ids.shape
    V, D = token_emb.shape
    mask_i32 = continuous_mask.astype(jnp.int32)
    gathered = _sc_gather(token_emb, token_ids.reshape(-1), mask_i32.reshape(-1))
    BB = 4
    pe_tiled = jnp.asarray(np.tile(_pe_table(S, D), (BB, 1)))
    out = _tc_fuse(
        gathered,
        token_ids.reshape(-1, 1).astype(jnp.float32),
        token_types.reshape(-1, 1),
        mask_i32.reshape(-1, 1),
        w1,
        b1.reshape(1, -1),
        w2,
        b2.reshape(1, -1),
        type_emb,
        pe_tiled,
        gamma.reshape(1, -1),
        beta.reshape(1, -1),
    )
    return out.reshape(B, S, D)


# speedup vs baseline: 6.8718x; 6.8718x over previous
import functools
import math

import jax
import jax.numpy as jnp
import numpy as np
from jax import lax
from jax.experimental import pallas as pl
from jax.experimental.pallas import tpu as pltpu
from jax.experimental.pallas import tpu_sc as plsc


def _pe_table(max_len, d):
    position = np.arange(max_len, dtype=np.float32)[:, None]
    div_term = np.exp(
        np.arange(0, d, 2, dtype=np.float32) * (-math.log(10000.0) / d)
    )
    pe = np.zeros((max_len, d), dtype=np.float32)
    pe[:, 0::2] = np.sin(position * div_term)
    pe[:, 1::2] = np.cos(position * div_term)
    return pe


def _sc_gather(table, ids_flat, mask_flat_i32):
    N = ids_flat.shape[0]
    V, D = table.shape
    info = plsc.get_sparse_core_info()
    NC, NS, L = info.num_cores, info.num_subcores, info.num_lanes
    NW = NC * NS
    assert N % NW == 0
    b_per_w = N // NW
    K = 16
    NBUF = 4
    assert K == L
    assert b_per_w % L == 0
    mesh = plsc.VectorSubcoreMesh(core_axis_name="c", subcore_axis_name="s")

    @functools.partial(
        pl.kernel,
        mesh=mesh,
        compiler_params=pltpu.CompilerParams(needs_layout_passes=False),
        out_type=jax.ShapeDtypeStruct((N + NW, D), jnp.float32),
        scratch_types=[
            pltpu.VMEM((b_per_w,), jnp.int32),
            pltpu.VMEM((b_per_w,), jnp.int32),
            pltpu.VMEM((b_per_w + 2 * L,), jnp.int32),
            pltpu.VMEM((b_per_w + 2 * L,), jnp.int32),
            [pltpu.VMEM((K, D), jnp.float32) for _ in range(NBUF)],
            [pltpu.SemaphoreType.DMA for _ in range(NBUF)],
            [pltpu.SemaphoreType.DMA for _ in range(NBUF)],
        ],
    )
    def gather_kernel(table_hbm, ids_hbm, mask_hbm, out_hbm, idx_v, msk_v,
                      cidx, cpos, rows, gsems, wsems):
        wid = lax.axis_index("s") * NC + lax.axis_index("c")
        base = wid * b_per_w
        pltpu.sync_copy(ids_hbm.at[pl.ds(base, b_per_w)], idx_v)
        pltpu.sync_copy(mask_hbm.at[pl.ds(base, b_per_w)], msk_v)

        iota = lax.iota(jnp.int32, L)
        trash = b_per_w + L + iota
        dn = lax.GatherDimensionNumbers(
            offset_dims=(), collapsed_slice_dims=(0,), start_index_map=(0,)
        )

        def permute(v, idx):
            return lax.gather(
                v, idx[:, None], dn, slice_sizes=(1,),
                mode=lax.GatherScatterMode.PROMISE_IN_BOUNDS,
            )

        def comp_body(i, off_v):
            iv = idx_v[pl.ds(i * L, L)]
            mv = msk_v[pl.ds(i * L, L)]
            keep = mv == 0
            s = 1 - mv
            for d in (1, 2, 4, 8):
                g = permute(s, jnp.maximum(iota - d, 0))
                s = s + jnp.where(iota >= d, g, 0)
            pc = permute(s, iota * 0 + (L - 1))
            ecs = s - (1 - mv)
            tgt = jnp.where(keep, off_v + ecs, trash)
            pos = base + i * L + iota
            plsc.store_scatter(cidx, [tgt], iv)
            plsc.store_scatter(cpos, [tgt], pos)
            return off_v + pc

        off_v = lax.fori_loop(
            0, b_per_w // L, comp_body, jnp.zeros((L,), jnp.int32)
        )
        m_cnt = off_v[0]
        cidx[pl.ds(m_cnt, L)] = jnp.zeros((L,), jnp.int32)
        cpos[pl.ds(m_cnt, L)] = jnp.zeros((L,), jnp.int32) + (N + wid)
        nch = (m_cnt + (K - 1)) // K

        def g_copy(c, b):
            return pltpu.make_async_copy(
                table_hbm.at[cidx.at[pl.ds(c * K, K)]], rows[b], gsems[b]
            )

        def w_copy(c, b):
            iv = cpos[pl.ds(c * K, K)]
            return pltpu.make_async_copy(rows[b], out_hbm.at[iv], wsems[b])

        for b in range(NBUF - 1):
            @pl.when(b < nch)
            def _():
                g_copy(b, b).start()

        def group(g, _):
            for b in range(NBUF):
                c = g * NBUF + b

                @pl.when(c < nch)
                def _():
                    g_copy(c, b).wait()
                    w_copy(c, b).start()
                    f = c + NBUF - 1
                    fb = (b + NBUF - 1) % NBUF

                    @pl.when(f < nch)
                    def _():
                        @pl.when(f >= NBUF)
                        def _():
                            w_copy(f - NBUF, fb).wait()

                        g_copy(f, fb).start()

            return 0

        lax.fori_loop(0, (nch + NBUF - 1) // NBUF, group, 0)
        for b in range(NBUF):
            c_l = nch - 1 - lax.rem(nch - 1 - b, NBUF)

            @pl.when((c_l >= 0) & (c_l < nch) & (c_l >= nch - NBUF))
            def _():
                w_copy(c_l, b).wait()

    return gather_kernel(table, ids_flat, mask_flat_i32)


def _tc_fuse(gathered, ids_col, types_col, mask_col, w1, b1, w2, b2, type_emb,
             pe_tiled, gamma, beta):
    N = ids_col.shape[0]
    D = gathered.shape[1]
    H = w1.shape[1]
    T = type_emb.shape[0]
    BLK = pe_tiled.shape[0]
    assert N % BLK == 0

    def body(g_ref, ids_ref, ty_ref, mk_ref, w1_ref, b1_ref, w2_ref, b2_ref,
             te_ref, pe_ref, ga_ref, be_ref, out_ref):
        ids = ids_ref[...]
        h = jnp.maximum(ids * w1_ref[...] + b1_ref[...], 0.0)
        cont = jnp.dot(h, w2_ref[...], preferred_element_type=jnp.float32) + b2_ref[...]
        ty = ty_ref[...]
        onehot = (ty == lax.broadcasted_iota(jnp.int32, (BLK, T), 1)).astype(jnp.float32)
        tvec = jnp.dot(onehot, te_ref[...], preferred_element_type=jnp.float32)
        mk = mk_ref[...]
        emb = jnp.where(mk != 0, cont, g_ref[...]) + tvec + pe_ref[...]
        mean = jnp.mean(emb, axis=-1, keepdims=True)
        var = jnp.mean(jnp.square(emb - mean), axis=-1, keepdims=True)
        out_ref[...] = (emb - mean) * lax.rsqrt(var + 1e-5) * ga_ref[...] + be_ref[...]

    grid = (N // BLK,)
    full = lambda shape: pl.BlockSpec(shape, lambda i: (0,) * len(shape))
    return pl.pallas_call(
        body,
        grid=grid,
        in_specs=[
            pl.BlockSpec((BLK, D), lambda i: (i, 0)),
            pl.BlockSpec((BLK, 1), lambda i: (i, 0)),
            pl.BlockSpec((BLK, 1), lambda i: (i, 0)),
            pl.BlockSpec((BLK, 1), lambda i: (i, 0)),
            full((1, H)),
            full((1, H)),
            full((H, D)),
            full((1, D)),
            full((T, D)),
            full((BLK, D)),
            full((1, D)),
            full((1, D)),
        ],
        out_specs=pl.BlockSpec((BLK, D), lambda i: (i, 0)),
        out_shape=jax.ShapeDtypeStruct((N, D), jnp.float32),
    )(gathered, ids_col, types_col, mask_col, w1, b1, w2, b2, type_emb,
      pe_tiled, gamma, beta)


def kernel(token_ids, token_types, continuous_mask, token_emb, w1, b1, w2, b2,
           type_emb, gamma, beta):
    B, S = token_ids.shape
    V, D = token_emb.shape
    mask_i32 = continuous_mask.astype(jnp.int32)
    gathered = _sc_gather(token_emb, token_ids.reshape(-1), mask_i32.reshape(-1))
    BB = 4
    pe_tiled = jnp.asarray(np.tile(_pe_table(S, D), (BB, 1)))
    out = _tc_fuse(
        gathered,
        token_ids.reshape(-1, 1).astype(jnp.float32),
        token_types.reshape(-1, 1),
        mask_i32.reshape(-1, 1),
        w1,
        b1.reshape(1, -1),
        w2,
        b2.reshape(1, -1),
        type_emb,
        pe_tiled,
        gamma.reshape(1, -1),
        beta.reshape(1, -1),
    )
    return out.reshape(B, S, D)


# speedup vs baseline: 7.6648x; 1.1154x over previous
import functools
import math

import jax
import jax.numpy as jnp
import numpy as np
from jax import lax
from jax.experimental import pallas as pl
from jax.experimental.pallas import tpu as pltpu
from jax.experimental.pallas import tpu_sc as plsc


def _pe_table(max_len, d):
    position = np.arange(max_len, dtype=np.float32)[:, None]
    div_term = np.exp(
        np.arange(0, d, 2, dtype=np.float32) * (-math.log(10000.0) / d)
    )
    pe = np.zeros((max_len, d), dtype=np.float32)
    pe[:, 0::2] = np.sin(position * div_term)
    pe[:, 1::2] = np.cos(position * div_term)
    return pe


def _sc_gather(table, ids_flat, mask_flat_i32):
    N = ids_flat.shape[0]
    V, D = table.shape
    info = plsc.get_sparse_core_info()
    NC, NS, L = info.num_cores, info.num_subcores, info.num_lanes
    NW = NC * NS
    assert N % NW == 0
    b_per_w = N // NW
    K = 16
    NBUF = 4
    assert K == L
    assert b_per_w % L == 0
    mesh = plsc.VectorSubcoreMesh(core_axis_name="c", subcore_axis_name="s")

    @functools.partial(
        pl.kernel,
        mesh=mesh,
        compiler_params=pltpu.CompilerParams(needs_layout_passes=False),
        out_type=jax.ShapeDtypeStruct((N + NW, D), jnp.float32),
        scratch_types=[
            pltpu.VMEM((b_per_w,), jnp.int32),
            pltpu.VMEM((b_per_w,), jnp.int32),
            pltpu.VMEM((b_per_w + 2 * L,), jnp.int32),
            pltpu.VMEM((b_per_w + 2 * L,), jnp.int32),
            [pltpu.VMEM((K, D), jnp.float32) for _ in range(NBUF)],
            [pltpu.SemaphoreType.DMA for _ in range(NBUF)],
            [pltpu.SemaphoreType.DMA for _ in range(NBUF)],
        ],
    )
    def gather_kernel(table_hbm, ids_hbm, mask_hbm, out_hbm, idx_v, msk_v,
                      cidx, cpos, rows, gsems, wsems):
        wid = lax.axis_index("s") * NC + lax.axis_index("c")
        base = wid * b_per_w
        pltpu.sync_copy(ids_hbm.at[pl.ds(base, b_per_w)], idx_v)
        pltpu.sync_copy(mask_hbm.at[pl.ds(base, b_per_w)], msk_v)

        iota = lax.iota(jnp.int32, L)
        trash = b_per_w + L + iota
        dn = lax.GatherDimensionNumbers(
            offset_dims=(), collapsed_slice_dims=(0,), start_index_map=(0,)
        )

        def permute(v, idx):
            return lax.gather(
                v, idx[:, None], dn, slice_sizes=(1,),
                mode=lax.GatherScatterMode.PROMISE_IN_BOUNDS,
            )

        def comp_body(i, off_v):
            iv = idx_v[pl.ds(i * L, L)]
            mv = msk_v[pl.ds(i * L, L)]
            keep = mv == 0
            s = 1 - mv
            for d in (1, 2, 4, 8):
                g = permute(s, jnp.maximum(iota - d, 0))
                s = s + jnp.where(iota >= d, g, 0)
            pc = permute(s, iota * 0 + (L - 1))
            ecs = s - (1 - mv)
            tgt = jnp.where(keep, off_v + ecs, trash)
            pos = base + i * L + iota
            plsc.store_scatter(cidx, [tgt], iv)
            plsc.store_scatter(cpos, [tgt], pos)
            return off_v + pc

        off_v = lax.fori_loop(
            0, b_per_w // L, comp_body, jnp.zeros((L,), jnp.int32)
        )
        m_cnt = off_v[0]
        cidx[pl.ds(m_cnt, L)] = jnp.zeros((L,), jnp.int32)
        cpos[pl.ds(m_cnt, L)] = jnp.zeros((L,), jnp.int32) + (N + wid)
        nch = (m_cnt + (K - 1)) // K

        def g_copy(c, b):
            return pltpu.make_async_copy(
                table_hbm.at[cidx.at[pl.ds(c * K, K)]], rows[b], gsems[b]
            )

        def w_copy(c, b):
            iv = cpos[pl.ds(c * K, K)]
            return pltpu.make_async_copy(rows[b], out_hbm.at[iv], wsems[b])

        for b in range(NBUF - 1):
            @pl.when(b < nch)
            def _():
                g_copy(b, b).start()

        def group(g, _):
            for b in range(NBUF):
                c = g * NBUF + b

                @pl.when(c < nch)
                def _():
                    g_copy(c, b).wait()
                    w_copy(c, b).start()
                    f = c + NBUF - 1
                    fb = (b + NBUF - 1) % NBUF

                    @pl.when(f < nch)
                    def _():
                        @pl.when(f >= NBUF)
                        def _():
                            w_copy(f - NBUF, fb).wait()

                        g_copy(f, fb).start()

            return 0

        lax.fori_loop(0, (nch + NBUF - 1) // NBUF, group, 0)
        for b in range(NBUF):
            c_l = nch - 1 - lax.rem(nch - 1 - b, NBUF)

            @pl.when((c_l >= 0) & (c_l < nch) & (c_l >= nch - NBUF))
            def _():
                w_copy(c_l, b).wait()

    return gather_kernel(table, ids_flat, mask_flat_i32)


def _tc_fuse(gathered, ids_col, types_col, mask_col, w1, b1, w2, b2, type_emb,
             pe_tiled, gamma, beta):
    N = ids_col.shape[0]
    D = gathered.shape[1]
    H = w1.shape[1]
    T = type_emb.shape[0]
    BLK = pe_tiled.shape[0]
    assert N % BLK == 0

    def body(g_ref, ids_ref, ty_ref, mk_ref, w1_ref, b1_ref, w2_ref, b2_ref,
             te_ref, pe_ref, ga_ref, be_ref, out_ref):
        ids = ids_ref[...]
        h = jnp.maximum(ids * w1_ref[...] + b1_ref[...], 0.0)
        cont = jnp.dot(h, w2_ref[...], preferred_element_type=jnp.float32) + b2_ref[...]
        ty = ty_ref[...]
        onehot = (ty == lax.broadcasted_iota(jnp.int32, (BLK, T), 1)).astype(jnp.float32)
        tvec = jnp.dot(onehot, te_ref[...], preferred_element_type=jnp.float32)
        mk = mk_ref[...]
        emb = jnp.where(mk != 0, cont, g_ref[...]) + tvec + pe_ref[...]
        mean = jnp.mean(emb, axis=-1, keepdims=True)
        var = jnp.mean(jnp.square(emb - mean), axis=-1, keepdims=True)
        out_ref[...] = (emb - mean) * lax.rsqrt(var + 1e-5) * ga_ref[...] + be_ref[...]

    grid = (N // BLK,)
    full = lambda shape: pl.BlockSpec(shape, lambda i: (0,) * len(shape))
    return pl.pallas_call(
        body,
        grid=grid,
        in_specs=[
            pl.BlockSpec((BLK, D), lambda i: (i, 0)),
            pl.BlockSpec((BLK, 1), lambda i: (i, 0)),
            pl.BlockSpec((BLK, 1), lambda i: (i, 0)),
            pl.BlockSpec((BLK, 1), lambda i: (i, 0)),
            full((1, H)),
            full((1, H)),
            full((H, D)),
            full((1, D)),
            full((T, D)),
            full((BLK, D)),
            full((1, D)),
            full((1, D)),
        ],
        out_specs=pl.BlockSpec((BLK, D), lambda i: (i, 0)),
        out_shape=jax.ShapeDtypeStruct((N, D), jnp.float32),
    )(gathered, ids_col, types_col, mask_col, w1, b1, w2, b2, type_emb,
      pe_tiled, gamma, beta)


def kernel(token_ids, token_types, continuous_mask, token_emb, w1, b1, w2, b2,
           type_emb, gamma, beta):
    B, S = token_ids.shape
    V, D = token_emb.shape
    mask_i32 = continuous_mask.astype(jnp.int32)
    gathered = _sc_gather(token_emb, token_ids.reshape(-1), mask_i32.reshape(-1))
    BB = 8
    pe_tiled = jnp.asarray(np.tile(_pe_table(S, D), (BB, 1)))
    out = _tc_fuse(
        gathered,
        token_ids.reshape(-1, 1).astype(jnp.float32),
        token_types.reshape(-1, 1),
        mask_i32.reshape(-1, 1),
        w1,
        b1.reshape(1, -1),
        w2,
        b2.reshape(1, -1),
        type_emb,
        pe_tiled,
        gamma.reshape(1, -1),
        beta.reshape(1, -1),
    )
    return out.reshape(B, S, D)


# speedup vs baseline: 7.9577x; 1.0382x over previous
import functools
import math

import jax
import jax.numpy as jnp
import numpy as np
from jax import lax
from jax.experimental import pallas as pl
from jax.experimental.pallas import tpu as pltpu
from jax.experimental.pallas import tpu_sc as plsc


def _pe_table(max_len, d):
    position = np.arange(max_len, dtype=np.float32)[:, None]
    div_term = np.exp(
        np.arange(0, d, 2, dtype=np.float32) * (-math.log(10000.0) / d)
    )
    pe = np.zeros((max_len, d), dtype=np.float32)
    pe[:, 0::2] = np.sin(position * div_term)
    pe[:, 1::2] = np.cos(position * div_term)
    return pe


def _sc_gather(table, ids_flat, mask_flat_i32):
    N = ids_flat.shape[0]
    V, D = table.shape
    info = plsc.get_sparse_core_info()
    NC, NS, L = info.num_cores, info.num_subcores, info.num_lanes
    NW = NC * NS
    assert N % NW == 0
    b_per_w = N // NW
    K = 16
    NBUF = 4
    assert K == L
    assert b_per_w % L == 0
    mesh = plsc.VectorSubcoreMesh(core_axis_name="c", subcore_axis_name="s")

    @functools.partial(
        pl.kernel,
        mesh=mesh,
        compiler_params=pltpu.CompilerParams(needs_layout_passes=False),
        out_type=jax.ShapeDtypeStruct((N + NW, D), jnp.float32),
        scratch_types=[
            pltpu.VMEM((b_per_w,), jnp.int32),
            pltpu.VMEM((b_per_w,), jnp.int32),
            pltpu.VMEM((b_per_w + 2 * L,), jnp.int32),
            pltpu.VMEM((b_per_w + 2 * L,), jnp.int32),
            [pltpu.VMEM((K, D), jnp.float32) for _ in range(NBUF)],
            [pltpu.SemaphoreType.DMA for _ in range(NBUF)],
            [pltpu.SemaphoreType.DMA for _ in range(NBUF)],
        ],
    )
    def gather_kernel(table_hbm, ids_hbm, mask_hbm, out_hbm, idx_v, msk_v,
                      cidx, cpos, rows, gsems, wsems):
        wid = lax.axis_index("s") * NC + lax.axis_index("c")
        base = wid * b_per_w
        pltpu.sync_copy(ids_hbm.at[pl.ds(base, b_per_w)], idx_v)
        pltpu.sync_copy(mask_hbm.at[pl.ds(base, b_per_w)], msk_v)

        iota = lax.iota(jnp.int32, L)
        trash = b_per_w + L + iota
        dn = lax.GatherDimensionNumbers(
            offset_dims=(), collapsed_slice_dims=(0,), start_index_map=(0,)
        )

        def permute(v, idx):
            return lax.gather(
                v, idx[:, None], dn, slice_sizes=(1,),
                mode=lax.GatherScatterMode.PROMISE_IN_BOUNDS,
            )

        def comp_body(i, off_v):
            iv = idx_v[pl.ds(i * L, L)]
            mv = msk_v[pl.ds(i * L, L)]
            keep = mv == 0
            s = 1 - mv
            for d in (1, 2, 4, 8):
                g = permute(s, jnp.maximum(iota - d, 0))
                s = s + jnp.where(iota >= d, g, 0)
            pc = permute(s, iota * 0 + (L - 1))
            ecs = s - (1 - mv)
            tgt = jnp.where(keep, off_v + ecs, trash)
            pos = base + i * L + iota
            plsc.store_scatter(cidx, [tgt], iv)
            plsc.store_scatter(cpos, [tgt], pos)
            return off_v + pc

        off_v = lax.fori_loop(
            0, b_per_w // L, comp_body, jnp.zeros((L,), jnp.int32)
        )
        m_cnt = off_v[0]
        cidx[pl.ds(m_cnt, L)] = jnp.zeros((L,), jnp.int32)
        cpos[pl.ds(m_cnt, L)] = jnp.zeros((L,), jnp.int32) + (N + wid)
        nch = (m_cnt + (K - 1)) // K

        def g_copy(c, b):
            return pltpu.make_async_copy(
                table_hbm.at[cidx.at[pl.ds(c * K, K)]], rows[b], gsems[b]
            )

        def w_copy(c, b):
            iv = cpos[pl.ds(c * K, K)]
            return pltpu.make_async_copy(rows[b], out_hbm.at[iv], wsems[b])

        for b in range(NBUF - 1):
            @pl.when(b < nch)
            def _():
                g_copy(b, b).start()

        def group(g, _):
            for b in range(NBUF):
                c = g * NBUF + b

                @pl.when(c < nch)
                def _():
                    g_copy(c, b).wait()
                    w_copy(c, b).start()
                    f = c + NBUF - 1
                    fb = (b + NBUF - 1) % NBUF

                    @pl.when(f < nch)
                    def _():
                        @pl.when(f >= NBUF)
                        def _():
                            w_copy(f - NBUF, fb).wait()

                        g_copy(f, fb).start()

            return 0

        lax.fori_loop(0, (nch + NBUF - 1) // NBUF, group, 0)
        for b in range(NBUF):
            c_l = nch - 1 - lax.rem(nch - 1 - b, NBUF)

            @pl.when((c_l >= 0) & (c_l < nch) & (c_l >= nch - NBUF))
            def _():
                w_copy(c_l, b).wait()

    return gather_kernel(table, ids_flat, mask_flat_i32)


def _tc_fuse(gathered, ids_col, types_col, mask_col, w1, b1, w2, b2, type_emb,
             pe_tiled, gamma, beta):
    N = ids_col.shape[0]
    D = gathered.shape[1]
    H = w1.shape[1]
    T = type_emb.shape[0]
    BLK = pe_tiled.shape[0]
    assert N % BLK == 0

    def body(g_ref, ids_ref, ty_ref, mk_ref, w1_ref, b1_ref, w2_ref, b2_ref,
             te_ref, pe_ref, ga_ref, be_ref, out_ref):
        ids = ids_ref[...]
        h = jnp.maximum(ids * w1_ref[...] + b1_ref[...], 0.0)
        cont = jnp.dot(h, w2_ref[...], preferred_element_type=jnp.float32) + b2_ref[...]
        ty = ty_ref[...]
        onehot = (ty == lax.broadcasted_iota(jnp.int32, (BLK, T), 1)).astype(jnp.float32)
        tvec = jnp.dot(onehot, te_ref[...], preferred_element_type=jnp.float32)
        mk = mk_ref[...]
        emb = jnp.where(mk != 0, cont, g_ref[...]) + tvec + pe_ref[...]
        mean = jnp.mean(emb, axis=-1, keepdims=True)
        var = jnp.mean(jnp.square(emb - mean), axis=-1, keepdims=True)
        out_ref[...] = (emb - mean) * lax.rsqrt(var + 1e-5) * ga_ref[...] + be_ref[...]

    grid = (N // BLK,)
    full = lambda shape: pl.BlockSpec(shape, lambda i: (0,) * len(shape))
    return pl.pallas_call(
        body,
        grid=grid,
        in_specs=[
            pl.BlockSpec((BLK, D), lambda i: (i, 0)),
            pl.BlockSpec((BLK, 1), lambda i: (i, 0)),
            pl.BlockSpec((BLK, 1), lambda i: (i, 0)),
            pl.BlockSpec((BLK, 1), lambda i: (i, 0)),
            full((1, H)),
            full((1, H)),
            full((H, D)),
            full((1, D)),
            full((T, D)),
            full((BLK, D)),
            full((1, D)),
            full((1, D)),
        ],
        out_specs=pl.BlockSpec((BLK, D), lambda i: (i, 0)),
        out_shape=jax.ShapeDtypeStruct((N, D), jnp.float32),
    )(gathered, ids_col, types_col, mask_col, w1, b1, w2, b2, type_emb,
      pe_tiled, gamma, beta)


def kernel(token_ids, token_types, continuous_mask, token_emb, w1, b1, w2, b2,
           type_emb, gamma, beta):
    B, S = token_ids.shape
    V, D = token_emb.shape
    mask_i32 = continuous_mask.astype(jnp.int32)
    gathered = _sc_gather(token_emb, token_ids.reshape(-1), mask_i32.reshape(-1))
    BB = 16
    pe_tiled = jnp.asarray(np.tile(_pe_table(S, D), (BB, 1)))
    out = _tc_fuse(
        gathered,
        token_ids.reshape(-1, 1).astype(jnp.float32),
        token_types.reshape(-1, 1),
        mask_i32.reshape(-1, 1),
        w1,
        b1.reshape(1, -1),
        w2,
        b2.reshape(1, -1),
        type_emb,
        pe_tiled,
        gamma.reshape(1, -1),
        beta.reshape(1, -1),
    )
    return out.reshape(B, S, D)
